# baseline (speedup 1.0000x reference)
import jax
import jax.numpy as jnp
from jax import lax
from jax.experimental import pallas as pl
from jax.experimental.pallas import tpu as pltpu
from jax.experimental.pallas import tpu_sc as plsc

NC = 2
NS = 16
NW = NC * NS
LANE = 16

VOCAB = 1000000
B = 16384
L = 16
SEM_D = 48
ETH_D = 16
D = 64

SCALE = 1.0 / L

TCOLS = 128
NBLK = 244
TAIL_BASE = NW * NBLK * TCOLS
NTAIL = 5


def _reformat_body(sem_t, eth_t, comb,
                   sbuf0, sbuf1, ebuf0, ebuf1, obuf, dsem0, dsem1, demit):
    wid = lax.axis_index("s") * NC + lax.axis_index("c")
    blk0 = wid * NBLK

    iota = lax.iota(jnp.int32, LANE)

    def stage(g, sbuf, ebuf, dsem):
        c = pl.multiple_of((blk0 + g) * TCOLS, TCOLS)
        pltpu.async_copy(sem_t.at[:, pl.ds(c, TCOLS)], sbuf, dsem)
        pltpu.async_copy(eth_t.at[:, pl.ds(c, TCOLS)], ebuf, dsem)

    def drain(sbuf, ebuf, dsem):
        pltpu.make_async_copy(sem_t.at[:, pl.ds(0, TCOLS)], sbuf, dsem).wait()
        pltpu.make_async_copy(eth_t.at[:, pl.ds(0, TCOLS)], ebuf, dsem).wait()

    def transpose_into(sbuf, ebuf, half):
        def col_fn(c, _):
            cv = jnp.broadcast_to(c, (LANE,))
            o = half * TCOLS + c
            for k in range(SEM_D // LANE):
                v = plsc.load_gather(sbuf, [iota + (k * LANE), cv])
                obuf[o, pl.ds(k * LANE, LANE)] = v
            v = plsc.load_gather(ebuf, [iota, cv])
            obuf[o, pl.ds(SEM_D, LANE)] = v
            return 0

        lax.fori_loop(0, TCOLS, col_fn, 0)

    stage(0, sbuf0, ebuf0, dsem0)
    stage(1, sbuf1, ebuf1, dsem1)

    def pair_fn(j, _):
        g = j * 2
        drain(sbuf0, ebuf0, dsem0)

        @pl.when(j > 0)
        def _():
            pltpu.make_async_copy(
                comb.at[pl.ds(0, 2 * TCOLS)], obuf, demit).wait()

        transpose_into(sbuf0, ebuf0, 0)
        stage(g + 2, sbuf0, ebuf0, dsem0)
        drain(sbuf1, ebuf1, dsem1)
        transpose_into(sbuf1, ebuf1, 1)
        stage(g + 3, sbuf1, ebuf1, dsem1)
        pltpu.async_copy(
            obuf, comb.at[pl.ds(pl.multiple_of((blk0 + g) * TCOLS, TCOLS),
                                2 * TCOLS)], demit)
        return 0

    lax.fori_loop(0, NBLK // 2, pair_fn, 0)
    drain(sbuf0, ebuf0, dsem0)
    drain(sbuf1, ebuf1, dsem1)
    pltpu.make_async_copy(comb.at[pl.ds(0, 2 * TCOLS)], obuf, demit).wait()

    @pl.when(wid < NTAIL)
    def _tail():
        c = jnp.minimum(TAIL_BASE + wid * TCOLS, VOCAB - TCOLS)
        c = pl.multiple_of(c, 64)
        pltpu.sync_copy(sem_t.at[:, pl.ds(c, TCOLS)], sbuf0)
        pltpu.sync_copy(eth_t.at[:, pl.ds(c, TCOLS)], ebuf0)
        transpose_into(sbuf0, ebuf0, 0)
        pltpu.sync_copy(obuf.at[pl.ds(0, TCOLS)], comb.at[pl.ds(c, TCOLS)])


BPW = B // NW
CHUNK = 128
NCH = BPW // CHUNK


def _lookup_body(idx_t, comb, pos_hbm, out_t,
                 idx_v, pos_v, pos_m, acc, out_c, sem_g):
    wid = lax.axis_index("s") * NC + lax.axis_index("c")
    base = wid * BPW

    pltpu.sync_copy(idx_t.at[:, pl.ds(base, BPW)], idx_v)
    pltpu.sync_copy(pos_hbm.at[pl.ds(0, L)], pos_v)
    for k in range(D // LANE):
        s = pos_v[0, pl.ds(k * LANE, LANE)]
        for r in range(1, L):
            s = s + pos_v[r, pl.ds(k * LANE, LANE)]
        pos_m[pl.ds(k * LANE, LANE)] = s * SCALE

    iota = lax.iota(jnp.int32, LANE)

    for c in range(NCH):
        pltpu.async_copy(
            comb.at[idx_v.at[0, pl.ds(c * CHUNK, CHUNK)]], acc, sem_g).wait()
        descs = []
        for l in range(1, L):
            descs.append(pltpu.async_copy(
                comb.at[idx_v.at[l, pl.ds(c * CHUNK, CHUNK)]], acc, sem_g,
                add=True))
        for dsc in descs:
            dsc.wait()

        def feat_fn(d, _):
            dv = jnp.broadcast_to(d, (LANE,))
            p = plsc.load_gather(pos_m, [dv])
            for k in range(CHUNK // LANE):
                v = plsc.load_gather(acc, [iota + (k * LANE), dv])
                out_c[d, pl.ds(k * LANE, LANE)] = v * SCALE + p
            return 0

        lax.fori_loop(0, D, feat_fn, 0)
        pltpu.sync_copy(out_c, out_t.at[:, pl.ds(base + c * CHUNK, CHUNK)])


def kernel(indices, semantic_table, ethical_table, pos_enc):
    idx_t = indices.astype(jnp.int32).T
    sem_t = semantic_table.T
    eth_t = ethical_table.T
    pos2d = pos_enc.reshape(pos_enc.shape[1], D)

    comb = pl.kernel(
        _reformat_body,
        out_type=jax.ShapeDtypeStruct((VOCAB, D), jnp.float32),
        mesh=plsc.VectorSubcoreMesh(core_axis_name="c", subcore_axis_name="s"),
        scratch_types=[
            pltpu.VMEM((SEM_D, TCOLS), jnp.float32),
            pltpu.VMEM((SEM_D, TCOLS), jnp.float32),
            pltpu.VMEM((ETH_D, TCOLS), jnp.float32),
            pltpu.VMEM((ETH_D, TCOLS), jnp.float32),
            pltpu.VMEM((2 * TCOLS, D), jnp.float32),
            pltpu.SemaphoreType.DMA,
            pltpu.SemaphoreType.DMA,
            pltpu.SemaphoreType.DMA,
        ],
        compiler_params=pltpu.CompilerParams(use_tc_tiling_on_sc=False, needs_layout_passes=False),
    )(sem_t, eth_t)

    out_t = pl.kernel(
        _lookup_body,
        out_type=jax.ShapeDtypeStruct((D, B), jnp.float32),
        mesh=plsc.VectorSubcoreMesh(core_axis_name="c", subcore_axis_name="s"),
        scratch_types=[
            pltpu.VMEM((L, BPW), jnp.int32),
            pltpu.VMEM((L, D), jnp.float32),
            pltpu.VMEM((D,), jnp.float32),
            pltpu.VMEM((CHUNK, D), jnp.float32),
            pltpu.VMEM((D, CHUNK), jnp.float32),
            pltpu.SemaphoreType.DMA,
        ],
        compiler_params=pltpu.CompilerParams(use_tc_tiling_on_sc=False, needs_layout_passes=False),
    )(idx_t, comb, pos2d)

    return out_t.T


# speedup vs baseline: 5.1990x; 5.1990x over previous
import jax
import jax.numpy as jnp
from jax import lax
from jax.experimental import pallas as pl
from jax.experimental.pallas import tpu as pltpu
from jax.experimental.pallas import tpu_sc as plsc

NC = 2
NS = 16
NW = NC * NS
LANE = 16

VOCAB = 1000000
B = 16384
L = 16
SEM_D = 48
ETH_D = 16
D = 64
CD = 128

SCALE = 1.0 / L

BC = 512
GRID = (VOCAB + BC - 1) // BC


def _combine_body(sem_ref, eth_ref, out_ref):
    both = jnp.concatenate([sem_ref[...], eth_ref[...]], axis=0)
    t = both.T
    out_ref[...] = jnp.concatenate(
        [t, jnp.zeros((BC, CD - D), jnp.float32)], axis=1)


BPW = B // NW
CHUNK = 128
NCH = BPW // CHUNK


def _lookup_body(idx_t, comb, pos_hbm, out_hbm,
                 idx_v, pos_v, acc, out_c, sem_g):
    wid = lax.axis_index("s") * NC + lax.axis_index("c")
    base = wid * BPW

    pltpu.sync_copy(idx_t.at[:, pl.ds(base, BPW)], idx_v)
    pltpu.sync_copy(pos_hbm.at[pl.ds(0, L)], pos_v)
    pos_m = []
    for k in range(D // LANE):
        s = pos_v[0, pl.ds(k * LANE, LANE)]
        for r in range(1, L):
            s = s + pos_v[r, pl.ds(k * LANE, LANE)]
        pos_m.append(s * SCALE)

    for c in range(NCH):
        pltpu.async_copy(
            comb.at[idx_v.at[0, pl.ds(c * CHUNK, CHUNK)]], acc, sem_g).wait()
        descs = []
        for l in range(1, L):
            descs.append(pltpu.async_copy(
                comb.at[idx_v.at[l, pl.ds(c * CHUNK, CHUNK)]], acc, sem_g,
                add=True))
        for dsc in descs:
            dsc.wait()

        def row_fn(r, _):
            for k in range(D // LANE):
                v = acc[r, pl.ds(k * LANE, LANE)]
                out_c[r, pl.ds(k * LANE, LANE)] = v * SCALE + pos_m[k]
            return 0

        lax.fori_loop(0, CHUNK, row_fn, 0)
        pltpu.sync_copy(out_c, out_hbm.at[pl.ds(base + c * CHUNK, CHUNK)])


def kernel(indices, semantic_table, ethical_table, pos_enc):
    idx_t = indices.astype(jnp.int32).T
    sem_t = semantic_table.T
    eth_t = ethical_table.T
    pos2d = pos_enc.reshape(pos_enc.shape[1], D)

    comb = pl.pallas_call(
        _combine_body,
        grid=(GRID,),
        in_specs=[
            pl.BlockSpec((SEM_D, BC), lambda j: (0, j)),
            pl.BlockSpec((ETH_D, BC), lambda j: (0, j)),
        ],
        out_specs=pl.BlockSpec((BC, CD), lambda j: (j, 0)),
        out_shape=jax.ShapeDtypeStruct((VOCAB, CD), jnp.float32),
    )(sem_t, eth_t)

    out = pl.kernel(
        _lookup_body,
        out_type=jax.ShapeDtypeStruct((B, D), jnp.float32),
        mesh=plsc.VectorSubcoreMesh(core_axis_name="c", subcore_axis_name="s"),
        scratch_types=[
            pltpu.VMEM((L, BPW), jnp.int32),
            pltpu.VMEM((L, D), jnp.float32),
            pltpu.VMEM((CHUNK, CD), jnp.float32),
            pltpu.VMEM((CHUNK, D), jnp.float32),
            pltpu.SemaphoreType.DMA,
        ],
    )(idx_t, comb, pos2d)

    return out


# speedup vs baseline: 8.4385x; 1.6231x over previous
import jax
import jax.numpy as jnp
from jax import lax
from jax.experimental import pallas as pl
from jax.experimental.pallas import tpu as pltpu
from jax.experimental.pallas import tpu_sc as plsc

NC = 2
NS = 16
NW = NC * NS
LANE = 16

VOCAB = 1000000
B = 16384
L = 16
SEM_D = 48
ETH_D = 16
D = 64
CD = 128

SCALE = 1.0 / L

BC = 1024
GRID = (VOCAB + BC - 1) // BC


def _combine_body(sem_ref, eth_ref, out_ref):
    both = jnp.concatenate([sem_ref[...], eth_ref[...]], axis=0)
    t = both.T
    out_ref[...] = jnp.concatenate(
        [t, jnp.zeros((BC, CD - D), jnp.float32)], axis=1)


BPW = B // NW
CHUNK = 128
NCH = BPW // CHUNK


def _lookup_body(idx_t, comb, pos_hbm, out_hbm,
                 idx_v, pos_v, acc, out_c, sem_g):
    wid = lax.axis_index("s") * NC + lax.axis_index("c")
    base = wid * BPW

    pltpu.sync_copy(idx_t.at[:, pl.ds(base, BPW)], idx_v)
    pltpu.sync_copy(pos_hbm.at[pl.ds(0, L)], pos_v)
    pos_m = []
    for k in range(D // LANE):
        s = pos_v[0, pl.ds(k * LANE, LANE)]
        for r in range(1, L):
            s = s + pos_v[r, pl.ds(k * LANE, LANE)]
        pos_m.append(s * SCALE)

    for c in range(NCH):
        pltpu.async_copy(
            comb.at[idx_v.at[0, pl.ds(c * CHUNK, CHUNK)]], acc, sem_g).wait()
        descs = []
        for l in range(1, L):
            descs.append(pltpu.async_copy(
                comb.at[idx_v.at[l, pl.ds(c * CHUNK, CHUNK)]], acc, sem_g,
                add=True))
        for dsc in descs:
            dsc.wait()

        def row_fn(r, _):
            for k in range(D // LANE):
                v = acc[r, pl.ds(k * LANE, LANE)]
                out_c[r, pl.ds(k * LANE, LANE)] = v * SCALE + pos_m[k]
            return 0

        lax.fori_loop(0, CHUNK, row_fn, 0)
        pltpu.sync_copy(out_c, out_hbm.at[pl.ds(base + c * CHUNK, CHUNK)])


def kernel(indices, semantic_table, ethical_table, pos_enc):
    idx_t = indices.astype(jnp.int32).T
    sem_t = semantic_table.T
    eth_t = ethical_table.T
    pos2d = pos_enc.reshape(pos_enc.shape[1], D)

    comb = pl.pallas_call(
        _combine_body,
        grid=(GRID,),
        in_specs=[
            pl.BlockSpec((SEM_D, BC), lambda j: (0, j)),
            pl.BlockSpec((ETH_D, BC), lambda j: (0, j)),
        ],
        out_specs=pl.BlockSpec((BC, CD), lambda j: (j, 0)),
        out_shape=jax.ShapeDtypeStruct((VOCAB, CD), jnp.float32),
    )(sem_t, eth_t)

    out = pl.kernel(
        _lookup_body,
        out_type=jax.ShapeDtypeStruct((B, D), jnp.float32),
        mesh=plsc.VectorSubcoreMesh(core_axis_name="c", subcore_axis_name="s"),
        scratch_types=[
            pltpu.VMEM((L, BPW), jnp.int32),
            pltpu.VMEM((L, D), jnp.float32),
            pltpu.VMEM((CHUNK, CD), jnp.float32),
            pltpu.VMEM((CHUNK, D), jnp.float32),
            pltpu.SemaphoreType.DMA,
        ],
    )(idx_t, comb, pos2d)

    return out


# speedup vs baseline: 11.8850x; 1.4084x over previous
import jax
import jax.numpy as jnp
from jax import lax
from jax.experimental import pallas as pl
from jax.experimental.pallas import tpu as pltpu
from jax.experimental.pallas import tpu_sc as plsc

NC = 2
NS = 16
NW = NC * NS
LANE = 16

VOCAB = 1000000
B = 16384
L = 16
SEM_D = 48
ETH_D = 16
D = 64
CD = 128

SCALE = 1.0 / L

BC = 2048
GRID = (VOCAB + BC - 1) // BC


def _combine_body(sem_ref, eth_ref, out_ref):
    both = jnp.concatenate([sem_ref[...], eth_ref[...]], axis=0)
    t = both.T
    out_ref[...] = jnp.concatenate(
        [t, jnp.zeros((BC, CD - D), jnp.float32)], axis=1)


BPW = B // NW
CHUNK = 128
NCH = BPW // CHUNK


def _lookup_body(idx_t, comb, pos_hbm, out_t,
                 idx_v, pos_v, pos_m, acc, out_c, sem_g):
    wid = lax.axis_index("s") * NC + lax.axis_index("c")
    base = wid * BPW

    pltpu.sync_copy(idx_t.at[:, pl.ds(base, BPW)], idx_v)
    pltpu.sync_copy(pos_hbm.at[pl.ds(0, L)], pos_v)
    for k in range(D // LANE):
        s = pos_v[0, pl.ds(k * LANE, LANE)]
        for r in range(1, L):
            s = s + pos_v[r, pl.ds(k * LANE, LANE)]
        pos_m[pl.ds(k * LANE, LANE)] = s * SCALE

    iota = lax.iota(jnp.int32, LANE)

    for c in range(NCH):
        pltpu.async_copy(
            comb.at[idx_v.at[0, pl.ds(c * CHUNK, CHUNK)]], acc, sem_g).wait()
        descs = []
        for l in range(1, L):
            descs.append(pltpu.async_copy(
                comb.at[idx_v.at[l, pl.ds(c * CHUNK, CHUNK)]], acc, sem_g,
                add=True))
        for dsc in descs:
            dsc.wait()

        def feat_fn(d, _):
            dv = jnp.broadcast_to(d, (LANE,))
            p = plsc.load_gather(pos_m, [dv])
            for k in range(CHUNK // LANE):
                v = plsc.load_gather(acc, [iota + (k * LANE), dv])
                out_c[d, pl.ds(k * LANE, LANE)] = v * SCALE + p
            return 0

        lax.fori_loop(0, D, feat_fn, 0)
        pltpu.sync_copy(out_c, out_t.at[:, pl.ds(base + c * CHUNK, CHUNK)])


def kernel(indices, semantic_table, ethical_table, pos_enc):
    idx_t = indices.astype(jnp.int32).T
    sem_t = semantic_table.T
    eth_t = ethical_table.T
    pos2d = pos_enc.reshape(pos_enc.shape[1], D)

    comb = pl.pallas_call(
        _combine_body,
        grid=(GRID,),
        in_specs=[
            pl.BlockSpec((SEM_D, BC), lambda j: (0, j)),
            pl.BlockSpec((ETH_D, BC), lambda j: (0, j)),
        ],
        out_specs=pl.BlockSpec((BC, CD), lambda j: (j, 0)),
        out_shape=jax.ShapeDtypeStruct((VOCAB, CD), jnp.float32),
    )(sem_t, eth_t)

    out_t = pl.kernel(
        _lookup_body,
        out_type=jax.ShapeDtypeStruct((D, B), jnp.float32),
        mesh=plsc.VectorSubcoreMesh(core_axis_name="c", subcore_axis_name="s"),
        scratch_types=[
            pltpu.VMEM((L, BPW), jnp.int32),
            pltpu.VMEM((L, D), jnp.float32),
            pltpu.VMEM((D,), jnp.float32),
            pltpu.VMEM((CHUNK, CD), jnp.float32),
            pltpu.VMEM((D, CHUNK), jnp.float32),
            pltpu.SemaphoreType.DMA,
        ],
        compiler_params=pltpu.CompilerParams(needs_layout_passes=False),
    )(idx_t, comb, pos2d)

    return out_t.T


# speedup vs baseline: 15.3324x; 1.2901x over previous
import jax
import jax.numpy as jnp
from jax import lax
from jax.experimental import pallas as pl
from jax.experimental.pallas import tpu as pltpu
from jax.experimental.pallas import tpu_sc as plsc

NC = 2
NS = 16
NW = NC * NS
LANE = 16

VOCAB = 1000000
B = 16384
L = 16
SEM_D = 48
ETH_D = 16
D = 64
CD = 128

SCALE = 1.0 / L

BC = 4096
GRID = (VOCAB + BC - 1) // BC


def _combine_body(sem_ref, eth_ref, out_ref):
    both = jnp.concatenate([sem_ref[...], eth_ref[...]], axis=0)
    t = both.T
    out_ref[...] = jnp.concatenate(
        [t, jnp.zeros((BC, CD - D), jnp.float32)], axis=1)


BPW = B // NW
CHUNK = 128
NCH = BPW // CHUNK


def _lookup_body(idx_t, comb, pos_hbm, out_t,
                 idx_v, pos_v, pos_m, acc, out_c, sem_g):
    wid = lax.axis_index("s") * NC + lax.axis_index("c")
    base = wid * BPW

    pltpu.sync_copy(idx_t.at[:, pl.ds(base, BPW)], idx_v)
    pltpu.sync_copy(pos_hbm.at[pl.ds(0, L)], pos_v)
    for k in range(D // LANE):
        s = pos_v[0, pl.ds(k * LANE, LANE)]
        for r in range(1, L):
            s = s + pos_v[r, pl.ds(k * LANE, LANE)]
        pos_m[pl.ds(k * LANE, LANE)] = s * SCALE

    iota = lax.iota(jnp.int32, LANE)

    for c in range(NCH):
        pltpu.async_copy(
            comb.at[idx_v.at[0, pl.ds(c * CHUNK, CHUNK)]], acc, sem_g).wait()
        descs = []
        for l in range(1, L):
            descs.append(pltpu.async_copy(
                comb.at[idx_v.at[l, pl.ds(c * CHUNK, CHUNK)]], acc, sem_g,
                add=True))
        for dsc in descs:
            dsc.wait()

        def feat_fn(d, _):
            dv = jnp.broadcast_to(d, (LANE,))
            p = plsc.load_gather(pos_m, [dv])
            for k in range(CHUNK // LANE):
                v = plsc.load_gather(acc, [iota + (k * LANE), dv])
                out_c[d, pl.ds(k * LANE, LANE)] = v * SCALE + p
            return 0

        lax.fori_loop(0, D, feat_fn, 0)
        pltpu.sync_copy(out_c, out_t.at[:, pl.ds(base + c * CHUNK, CHUNK)])


def kernel(indices, semantic_table, ethical_table, pos_enc):
    idx_t = indices.astype(jnp.int32).T
    sem_t = semantic_table.T
    eth_t = ethical_table.T
    pos2d = pos_enc.reshape(pos_enc.shape[1], D)

    comb = pl.pallas_call(
        _combine_body,
        grid=(GRID,),
        in_specs=[
            pl.BlockSpec((SEM_D, BC), lambda j: (0, j)),
            pl.BlockSpec((ETH_D, BC), lambda j: (0, j)),
        ],
        out_specs=pl.BlockSpec((BC, CD), lambda j: (j, 0)),
        out_shape=jax.ShapeDtypeStruct((VOCAB, CD), jnp.float32),
    )(sem_t, eth_t)

    out_t = pl.kernel(
        _lookup_body,
        out_type=jax.ShapeDtypeStruct((D, B), jnp.float32),
        mesh=plsc.VectorSubcoreMesh(core_axis_name="c", subcore_axis_name="s"),
        scratch_types=[
            pltpu.VMEM((L, BPW), jnp.int32),
            pltpu.VMEM((L, D), jnp.float32),
            pltpu.VMEM((D,), jnp.float32),
            pltpu.VMEM((CHUNK, CD), jnp.float32),
            pltpu.VMEM((D, CHUNK), jnp.float32),
            pltpu.SemaphoreType.DMA,
        ],
        compiler_params=pltpu.CompilerParams(needs_layout_passes=False),
    )(idx_t, comb, pos2d)

    return out_t.T


# speedup vs baseline: 18.3779x; 1.1986x over previous
import jax
import jax.numpy as jnp
from jax import lax
from jax.experimental import pallas as pl
from jax.experimental.pallas import tpu as pltpu
from jax.experimental.pallas import tpu_sc as plsc

NC = 2
NS = 16
NW = NC * NS
LANE = 16

VOCAB = 1000000
B = 16384
L = 16
SEM_D = 48
ETH_D = 16
D = 64
CD = 128

SCALE = 1.0 / L

BC = 8192
GRID = (VOCAB + BC - 1) // BC


def _combine_body(sem_ref, eth_ref, out_ref):
    both = jnp.concatenate([sem_ref[...], eth_ref[...]], axis=0)
    t = both.T
    out_ref[...] = jnp.concatenate(
        [t, jnp.zeros((BC, CD - D), jnp.float32)], axis=1)


BPW = B // NW
CHUNK = 128
NCH = BPW // CHUNK


def _lookup_body(idx_t, comb, pos_hbm, out_t,
                 idx_v, pos_v, pos_m, acc, out_c, sem_g):
    wid = lax.axis_index("s") * NC + lax.axis_index("c")
    base = wid * BPW

    pltpu.sync_copy(idx_t.at[:, pl.ds(base, BPW)], idx_v)
    pltpu.sync_copy(pos_hbm.at[pl.ds(0, L)], pos_v)
    for k in range(D // LANE):
        s = pos_v[0, pl.ds(k * LANE, LANE)]
        for r in range(1, L):
            s = s + pos_v[r, pl.ds(k * LANE, LANE)]
        pos_m[pl.ds(k * LANE, LANE)] = s * SCALE

    iota = lax.iota(jnp.int32, LANE)

    for c in range(NCH):
        pltpu.async_copy(
            comb.at[idx_v.at[0, pl.ds(c * CHUNK, CHUNK)]], acc, sem_g).wait()
        descs = []
        for l in range(1, L):
            descs.append(pltpu.async_copy(
                comb.at[idx_v.at[l, pl.ds(c * CHUNK, CHUNK)]], acc, sem_g,
                add=True))
        for dsc in descs:
            dsc.wait()

        def feat_fn(d, _):
            dv = jnp.broadcast_to(d, (LANE,))
            p = plsc.load_gather(pos_m, [dv])
            for k in range(CHUNK // LANE):
                v = plsc.load_gather(acc, [iota + (k * LANE), dv])
                out_c[d, pl.ds(k * LANE, LANE)] = v * SCALE + p
            return 0

        lax.fori_loop(0, D, feat_fn, 0)
        pltpu.sync_copy(out_c, out_t.at[:, pl.ds(base + c * CHUNK, CHUNK)])


def kernel(indices, semantic_table, ethical_table, pos_enc):
    idx_t = indices.astype(jnp.int32).T
    sem_t = semantic_table.T
    eth_t = ethical_table.T
    pos2d = pos_enc.reshape(pos_enc.shape[1], D)

    comb = pl.pallas_call(
        _combine_body,
        grid=(GRID,),
        in_specs=[
            pl.BlockSpec((SEM_D, BC), lambda j: (0, j)),
            pl.BlockSpec((ETH_D, BC), lambda j: (0, j)),
        ],
        out_specs=pl.BlockSpec((BC, CD), lambda j: (j, 0)),
        out_shape=jax.ShapeDtypeStruct((VOCAB, CD), jnp.float32),
    )(sem_t, eth_t)

    out_t = pl.kernel(
        _lookup_body,
        out_type=jax.ShapeDtypeStruct((D, B), jnp.float32),
        mesh=plsc.VectorSubcoreMesh(core_axis_name="c", subcore_axis_name="s"),
        scratch_types=[
            pltpu.VMEM((L, BPW), jnp.int32),
            pltpu.VMEM((L, D), jnp.float32),
            pltpu.VMEM((D,), jnp.float32),
            pltpu.VMEM((CHUNK, CD), jnp.float32),
            pltpu.VMEM((D, CHUNK), jnp.float32),
            pltpu.SemaphoreType.DMA,
        ],
        compiler_params=pltpu.CompilerParams(needs_layout_passes=False),
    )(idx_t, comb, pos2d)

    return out_t.T


# speedup vs baseline: 19.3668x; 1.0538x over previous
import jax
import jax.numpy as jnp
from jax import lax
from jax.experimental import pallas as pl
from jax.experimental.pallas import tpu as pltpu
from jax.experimental.pallas import tpu_sc as plsc

NC = 2
NS = 16
NW = NC * NS
LANE = 16

VOCAB = 1000000
B = 16384
L = 16
SEM_D = 48
ETH_D = 16
D = 64
CD = 128

SCALE = 1.0 / L

BC = 16384
GRID = (VOCAB + BC - 1) // BC


def _combine_body(sem_ref, eth_ref, out_ref):
    both = jnp.concatenate([sem_ref[...], eth_ref[...]], axis=0)
    t = both.T
    out_ref[...] = jnp.concatenate(
        [t, jnp.zeros((BC, CD - D), jnp.float32)], axis=1)


BPW = B // NW
CHUNK = 128
NCH = BPW // CHUNK


def _lookup_body(idx_t, comb, pos_hbm, out_t,
                 idx_v, pos_v, pos_m, acc, out_c, sem_g):
    wid = lax.axis_index("s") * NC + lax.axis_index("c")
    base = wid * BPW

    pltpu.sync_copy(idx_t.at[:, pl.ds(base, BPW)], idx_v)
    pltpu.sync_copy(pos_hbm.at[pl.ds(0, L)], pos_v)
    for k in range(D // LANE):
        s = pos_v[0, pl.ds(k * LANE, LANE)]
        for r in range(1, L):
            s = s + pos_v[r, pl.ds(k * LANE, LANE)]
        pos_m[pl.ds(k * LANE, LANE)] = s * SCALE

    iota = lax.iota(jnp.int32, LANE)

    for c in range(NCH):
        pltpu.async_copy(
            comb.at[idx_v.at[0, pl.ds(c * CHUNK, CHUNK)]], acc, sem_g).wait()
        descs = []
        for l in range(1, L):
            descs.append(pltpu.async_copy(
                comb.at[idx_v.at[l, pl.ds(c * CHUNK, CHUNK)]], acc, sem_g,
                add=True))
        for dsc in descs:
            dsc.wait()

        def feat_fn(d, _):
            dv = jnp.broadcast_to(d, (LANE,))
            p = plsc.load_gather(pos_m, [dv])
            for k in range(CHUNK // LANE):
                v = plsc.load_gather(acc, [iota + (k * LANE), dv])
                out_c[d, pl.ds(k * LANE, LANE)] = v * SCALE + p
            return 0

        lax.fori_loop(0, D, feat_fn, 0)
        pltpu.sync_copy(out_c, out_t.at[:, pl.ds(base + c * CHUNK, CHUNK)])


def kernel(indices, semantic_table, ethical_table, pos_enc):
    idx_t = indices.astype(jnp.int32).T
    sem_t = semantic_table.T
    eth_t = ethical_table.T
    pos2d = pos_enc.reshape(pos_enc.shape[1], D)

    comb = pl.pallas_call(
        _combine_body,
        grid=(GRID,),
        in_specs=[
            pl.BlockSpec((SEM_D, BC), lambda j: (0, j)),
            pl.BlockSpec((ETH_D, BC), lambda j: (0, j)),
        ],
        out_specs=pl.BlockSpec((BC, CD), lambda j: (j, 0)),
        out_shape=jax.ShapeDtypeStruct((VOCAB, CD), jnp.float32),
    )(sem_t, eth_t)

    out_t = pl.kernel(
        _lookup_body,
        out_type=jax.ShapeDtypeStruct((D, B), jnp.float32),
        mesh=plsc.VectorSubcoreMesh(core_axis_name="c", subcore_axis_name="s"),
        scratch_types=[
            pltpu.VMEM((L, BPW), jnp.int32),
            pltpu.VMEM((L, D), jnp.float32),
            pltpu.VMEM((D,), jnp.float32),
            pltpu.VMEM((CHUNK, CD), jnp.float32),
            pltpu.VMEM((D, CHUNK), jnp.float32),
            pltpu.SemaphoreType.DMA,
        ],
        compiler_params=pltpu.CompilerParams(needs_layout_passes=False),
    )(idx_t, comb, pos2d)

    return out_t.T


# speedup vs baseline: 20.2137x; 1.0437x over previous
import jax
import jax.numpy as jnp
from jax import lax
from jax.experimental import pallas as pl
from jax.experimental.pallas import tpu as pltpu
from jax.experimental.pallas import tpu_sc as plsc

NC = 2
NS = 16
NW = NC * NS
LANE = 16

VOCAB = 1000000
B = 16384
L = 16
SEM_D = 48
ETH_D = 16
D = 64
CD = 128

SCALE = 1.0 / L

BC = 16384
GRID = (VOCAB + BC - 1) // BC


def _combine_body(sem_ref, eth_ref, out_ref):
    both = jnp.concatenate([sem_ref[...], eth_ref[...]], axis=0)
    t = both.T
    out_ref[...] = jnp.concatenate(
        [t, jnp.zeros((BC, CD - D), jnp.float32)], axis=1)


BPW = B // NW
CHUNK = 128
NCH = BPW // CHUNK


def _lookup_body(idx_t, comb, pos_hbm, out_t,
                 idx_v, pos_v, pos_m, acc0, acc1, out_c, sg0, sg1):
    wid = lax.axis_index("s") * NC + lax.axis_index("c")
    base = wid * BPW

    pltpu.sync_copy(idx_t.at[:, pl.ds(base, BPW)], idx_v)
    pltpu.sync_copy(pos_hbm.at[pl.ds(0, L)], pos_v)
    for k in range(D // LANE):
        s = pos_v[0, pl.ds(k * LANE, LANE)]
        for r in range(1, L):
            s = s + pos_v[r, pl.ds(k * LANE, LANE)]
        pos_m[pl.ds(k * LANE, LANE)] = s * SCALE

    iota = lax.iota(jnp.int32, LANE)
    accs = (acc0, acc1)
    sems = (sg0, sg1)
    zvec = jnp.zeros((LANE,), jnp.float32)

    def zero(acc):
        def zr(r, _):
            for k in range(D // LANE):
                acc[r, pl.ds(k * LANE, LANE)] = zvec
            return 0

        lax.fori_loop(0, CHUNK, zr, 0)

    def fire(c, p):
        for l in range(L):
            pltpu.async_copy(
                comb.at[idx_v.at[l, pl.ds(c * CHUNK, CHUNK)]], accs[p],
                sems[p], add=True)

    def drain(p):
        for _ in range(L):
            pltpu.make_async_copy(
                comb.at[idx_v.at[0, pl.ds(0, CHUNK)]], accs[p],
                sems[p]).wait()

    def post(c, p):
        acc = accs[p]

        def feat_fn(d, _):
            dv = jnp.broadcast_to(d, (LANE,))
            pv = plsc.load_gather(pos_m, [dv])
            for k in range(CHUNK // LANE):
                v = plsc.load_gather(acc, [iota + (k * LANE), dv])
                out_c[d, pl.ds(k * LANE, LANE)] = v * SCALE + pv
            return 0

        lax.fori_loop(0, D, feat_fn, 0)
        pltpu.sync_copy(out_c, out_t.at[:, pl.ds(base + c * CHUNK, CHUNK)])

    zero(accs[0])
    fire(0, 0)
    for c in range(NCH):
        p = c % 2
        if c + 1 < NCH:
            zero(accs[1 - p])
            fire(c + 1, 1 - p)
        drain(p)
        post(c, p)


def kernel(indices, semantic_table, ethical_table, pos_enc):
    idx_t = indices.astype(jnp.int32).T
    sem_t = semantic_table.T
    eth_t = ethical_table.T
    pos2d = pos_enc.reshape(pos_enc.shape[1], D)

    comb = pl.pallas_call(
        _combine_body,
        grid=(GRID,),
        in_specs=[
            pl.BlockSpec((SEM_D, BC), lambda j: (0, j)),
            pl.BlockSpec((ETH_D, BC), lambda j: (0, j)),
        ],
        out_specs=pl.BlockSpec((BC, CD), lambda j: (j, 0)),
        out_shape=jax.ShapeDtypeStruct((VOCAB, CD), jnp.float32),
    )(sem_t, eth_t)

    out_t = pl.kernel(
        _lookup_body,
        out_type=jax.ShapeDtypeStruct((D, B), jnp.float32),
        mesh=plsc.VectorSubcoreMesh(core_axis_name="c", subcore_axis_name="s"),
        scratch_types=[
            pltpu.VMEM((L, BPW), jnp.int32),
            pltpu.VMEM((L, D), jnp.float32),
            pltpu.VMEM((D,), jnp.float32),
            pltpu.VMEM((CHUNK, CD), jnp.float32),
            pltpu.VMEM((CHUNK, CD), jnp.float32),
            pltpu.VMEM((D, CHUNK), jnp.float32),
            pltpu.SemaphoreType.DMA,
            pltpu.SemaphoreType.DMA,
        ],
        compiler_params=pltpu.CompilerParams(needs_layout_passes=False),
    )(idx_t, comb, pos2d)

    return out_t.T
